# Initial kernel scaffold; baseline (speedup 1.0000x reference)
import jax
import jax.numpy as jnp
from jax.experimental import pallas as pl


def kernel(x, edge_index, W1, b1, W2, b2):
    raise NotImplementedError("write your pallas kernel here")


# speedup vs baseline: 16.8911x; 16.8911x over previous
import functools

import jax
import jax.numpy as jnp
from jax import lax
from jax.experimental import pallas as pl
from jax.experimental.pallas import tpu as pltpu
from jax.experimental.pallas import tpu_sc as plsc

_N = 10000
_E = 320000
_D_IN = 128
_D_H = 64

_NC = 2
_NS = 16
_NW = _NC * _NS
_CHUNK = 128

_NPAD = 10240
_RPT = _NPAD // _NS
_EPW = 10240
_E_PAD = _EPW * _NW
_NCHUNK = _EPW // _CHUNK
_DEG_W = 16


def _sc_degree(dstp):
  mesh = plsc.VectorSubcoreMesh(core_axis_name="c", subcore_axis_name="s")

  @functools.partial(
      pl.kernel,
      out_type=jax.ShapeDtypeStruct((_NC * _NPAD, _DEG_W), jnp.float32),
      mesh=mesh,
      scratch_types=[
          pltpu.VMEM((_CHUNK,), jnp.int32),
          pltpu.VMEM((_CHUNK, _DEG_W), jnp.float32),
          pltpu.VMEM_SHARED((_NPAD, _DEG_W), jnp.float32),
      ],
      compiler_params=pltpu.CompilerParams(use_tc_tiling_on_sc=False),
  )
  def k(d_hbm, out_hbm, didx, buf, acc):
    cid = lax.axis_index("c")
    sid = lax.axis_index("s")
    wid = sid * _NC + cid
    r0 = sid * _RPT

    def fill(i, val):
      buf[i] = jnp.full((_DEG_W,), val, jnp.float32)
      return val

    lax.fori_loop(0, _CHUNK, fill, 0.0)
    for j in range(_RPT // _CHUNK):
      pltpu.sync_copy(buf, acc.at[pl.ds(r0 + j * _CHUNK, _CHUNK)])
    lax.fori_loop(0, _CHUNK, fill, 1.0)
    plsc.subcore_barrier()

    base = wid * _EPW

    def body(i, carry):
      pltpu.sync_copy(d_hbm.at[pl.ds(base + i * _CHUNK, _CHUNK)], didx)
      pltpu.sync_copy(buf, acc.at[didx], add=True)
      return carry

    lax.fori_loop(0, _NCHUNK, body, 0)
    plsc.subcore_barrier()
    pltpu.sync_copy(acc.at[pl.ds(r0, _RPT)],
                    out_hbm.at[pl.ds(cid * _NPAD + r0, _RPT)])

  return k(dstp)


def _sc_edge_pass(z, srcp, dstp):
  mesh = plsc.VectorSubcoreMesh(core_axis_name="c", subcore_axis_name="s")

  @functools.partial(
      pl.kernel,
      out_type=jax.ShapeDtypeStruct((_NC * _NPAD, _D_H), jnp.float32),
      mesh=mesh,
      scratch_types=[
          pltpu.VMEM((_CHUNK,), jnp.int32),
          pltpu.VMEM((_CHUNK,), jnp.int32),
          pltpu.VMEM((_CHUNK, _D_H), jnp.float32),
          pltpu.VMEM_SHARED((_NPAD, _D_H), jnp.float32),
          pltpu.SemaphoreType.DMA,
      ],
      compiler_params=pltpu.CompilerParams(use_tc_tiling_on_sc=False),
  )
  def k(z_hbm, s_hbm, d_hbm, out_hbm, sidx, didx, rows, acc, sem):
    cid = lax.axis_index("c")
    sid = lax.axis_index("s")
    wid = sid * _NC + cid
    r0 = sid * _RPT

    def zfill(i, carry):
      r = i // (_D_H // 16)
      c = (i % (_D_H // 16)) * 16
      rows[r, pl.ds(c, 16)] = jnp.zeros((16,), jnp.float32)
      return carry

    lax.fori_loop(0, _CHUNK * (_D_H // 16), zfill, 0)
    for j in range(_RPT // _CHUNK):
      pltpu.sync_copy(rows, acc.at[pl.ds(r0 + j * _CHUNK, _CHUNK)])
    plsc.subcore_barrier()

    base = wid * _EPW

    def body(i, carry):
      off = base + i * _CHUNK
      pltpu.sync_copy(s_hbm.at[pl.ds(off, _CHUNK)], sidx)
      pltpu.sync_copy(d_hbm.at[pl.ds(off, _CHUNK)], didx)
      pltpu.async_copy(z_hbm.at[sidx], rows, sem).wait()
      pltpu.sync_copy(rows, acc.at[didx], add=True)
      return carry

    lax.fori_loop(0, _NCHUNK, body, 0)
    plsc.subcore_barrier()
    pltpu.sync_copy(acc.at[pl.ds(r0, _RPT)],
                    out_hbm.at[pl.ds(cid * _NPAD + r0, _RPT)])

  return k(z, srcp, dstp)


_BR = 1024


def _tc_layer1(xp, W1, deg0, deg1):
  def body(x_ref, w_ref, d0_ref, d1_ref, z_ref, dv_ref):
    pid = pl.program_id(0)
    deg = d0_ref[:, 0:1] + d1_ref[:, 0:1] + 1.0
    rows = lax.broadcasted_iota(jnp.int32, (_BR, 1), 0) + pid * _BR
    dinv = jnp.where(rows < _N, 1.0 / jnp.sqrt(deg), 0.0)
    xw = jnp.dot(x_ref[...], w_ref[...], preferred_element_type=jnp.float32)
    z_ref[...] = dinv * xw
    dv_ref[...] = dinv

  return pl.pallas_call(
      body,
      grid=(_NPAD // _BR,),
      in_specs=[
          pl.BlockSpec((_BR, _D_IN), lambda i: (i, 0)),
          pl.BlockSpec((_D_IN, _D_H), lambda i: (0, 0)),
          pl.BlockSpec((_BR, _DEG_W), lambda i: (i, 0)),
          pl.BlockSpec((_BR, _DEG_W), lambda i: (i, 0)),
      ],
      out_specs=[
          pl.BlockSpec((_BR, _D_H), lambda i: (i, 0)),
          pl.BlockSpec((_BR, 1), lambda i: (i, 0)),
      ],
      out_shape=[
          jax.ShapeDtypeStruct((_NPAD, _D_H), jnp.float32),
          jax.ShapeDtypeStruct((_NPAD, 1), jnp.float32),
      ],
  )(xp, W1, deg0, deg1)


def _tc_mid(a0, a1, z1, dinv, b1, W2):
  def body(a0_ref, a1_ref, z_ref, dv_ref, b_ref, w_ref, z2_ref):
    dv = dv_ref[...]
    h = jnp.maximum(
        dv * (a0_ref[...] + a1_ref[...] + z_ref[...]) + b_ref[...], 0.0)
    z2_ref[...] = dv * jnp.dot(h, w_ref[...],
                               preferred_element_type=jnp.float32)

  return pl.pallas_call(
      body,
      grid=(_NPAD // _BR,),
      in_specs=[
          pl.BlockSpec((_BR, _D_H), lambda i: (i, 0)),
          pl.BlockSpec((_BR, _D_H), lambda i: (i, 0)),
          pl.BlockSpec((_BR, _D_H), lambda i: (i, 0)),
          pl.BlockSpec((_BR, 1), lambda i: (i, 0)),
          pl.BlockSpec((1, _D_H), lambda i: (0, 0)),
          pl.BlockSpec((_D_H, _D_H), lambda i: (0, 0)),
      ],
      out_specs=pl.BlockSpec((_BR, _D_H), lambda i: (i, 0)),
      out_shape=jax.ShapeDtypeStruct((_NPAD, _D_H), jnp.float32),
  )(a0, a1, z1, dinv, b1, W2)


def _tc_final(a0, a1, z2, dinv, b2):
  def body(a0_ref, a1_ref, z_ref, dv_ref, b_ref, o_ref):
    o_ref[...] = dv_ref[...] * (
        a0_ref[...] + a1_ref[...] + z_ref[...]) + b_ref[...]

  return pl.pallas_call(
      body,
      grid=(_NPAD // _BR,),
      in_specs=[
          pl.BlockSpec((_BR, _D_H), lambda i: (i, 0)),
          pl.BlockSpec((_BR, _D_H), lambda i: (i, 0)),
          pl.BlockSpec((_BR, _D_H), lambda i: (i, 0)),
          pl.BlockSpec((_BR, 1), lambda i: (i, 0)),
          pl.BlockSpec((1, _D_H), lambda i: (0, 0)),
      ],
      out_specs=pl.BlockSpec((_BR, _D_H), lambda i: (i, 0)),
      out_shape=jax.ShapeDtypeStruct((_NPAD, _D_H), jnp.float32),
  )(a0, a1, z2, dinv, b2)


def kernel(x, edge_index, W1, b1, W2, b2):
  src = edge_index[0]
  dst = edge_index[1]

  xp = jnp.pad(x, ((0, _NPAD - _N), (0, 0)))
  pad = _E_PAD - _E
  pad_rows = _N + (jnp.arange(pad, dtype=jnp.int32) % (_NPAD - _N))
  srcp = jnp.concatenate([src, pad_rows])
  dstp = jnp.concatenate([dst, pad_rows])

  degp = _sc_degree(dstp).reshape(_NC, _NPAD, _DEG_W)
  z1, dinv = _tc_layer1(xp, W1, degp[0], degp[1])

  acc1 = _sc_edge_pass(z1, srcp, dstp).reshape(_NC, _NPAD, _D_H)
  z2 = _tc_mid(acc1[0], acc1[1], z1, dinv, b1.reshape(1, _D_H), W2)

  acc2 = _sc_edge_pass(z2, srcp, dstp).reshape(_NC, _NPAD, _D_H)
  out = _tc_final(acc2[0], acc2[1], z2, dinv, b2.reshape(1, _D_H))
  return out[:_N]


# speedup vs baseline: 39.2639x; 2.3245x over previous
import functools

import jax
import jax.numpy as jnp
from jax import lax
from jax.experimental import pallas as pl
from jax.experimental.pallas import tpu as pltpu
from jax.experimental.pallas import tpu_sc as plsc

_N = 10000
_E = 320000
_D_IN = 128
_D_H = 64

_NC = 2
_NS = 16
_NW = _NC * _NS
_CHUNK = 128

_NPAD = 10240
_RPT = _NPAD // _NS
_EPW = 10240
_E_PAD = _EPW * _NW
_NCHUNK = _EPW // _CHUNK
_DEG_W = 16
_NB = 4

_SC_PARAMS = pltpu.CompilerParams(use_tc_tiling_on_sc=False)


def _sc_degree(dst2d):
  mesh = plsc.VectorSubcoreMesh(core_axis_name="c", subcore_axis_name="s")

  @functools.partial(
      pl.kernel,
      out_type=jax.ShapeDtypeStruct((_NC * _NPAD, _DEG_W), jnp.float32),
      mesh=mesh,
      scratch_types=[
          pltpu.VMEM((_NCHUNK, _CHUNK), jnp.int32),
          pltpu.VMEM((_CHUNK, _DEG_W), jnp.float32),
          pltpu.VMEM_SHARED((_NPAD, _DEG_W), jnp.float32),
          pltpu.SemaphoreType.DMA,
          pltpu.SemaphoreType.DMA,
      ],
      compiler_params=_SC_PARAMS,
  )
  def k(d_hbm, out_hbm, didx, buf, acc, isem, ssem):
    cid = lax.axis_index("c")
    sid = lax.axis_index("s")
    wid = sid * _NC + cid
    r0 = sid * _RPT

    idxc = pltpu.async_copy(
        d_hbm.at[pl.ds(wid * _NCHUNK, _NCHUNK)], didx, isem)

    def fill(i, val):
      buf[i] = jnp.full((_DEG_W,), val, jnp.float32)
      return val

    lax.fori_loop(0, _CHUNK, fill, 0.0)
    zcs = [
        pltpu.async_copy(buf, acc.at[pl.ds(r0 + j * _CHUNK, _CHUNK)], ssem)
        for j in range(_RPT // _CHUNK)
    ]
    for c in zcs:
      c.wait()
    lax.fori_loop(0, _CHUNK, fill, 1.0)
    idxc.wait()
    plsc.subcore_barrier()

    def body(t, carry):
      cs = [
          pltpu.async_copy(buf, acc.at[didx.at[8 * t + b]], ssem, add=True)
          for b in range(8)
      ]
      for c in cs:
        c.wait()
      return carry

    lax.fori_loop(0, _NCHUNK // 8, body, 0)
    plsc.subcore_barrier()
    pltpu.sync_copy(acc.at[pl.ds(r0, _RPT)],
                    out_hbm.at[pl.ds(cid * _NPAD + r0, _RPT)])

  return k(dst2d)


def _sc_edge_pass(z, src2d, dst2d):
  mesh = plsc.VectorSubcoreMesh(core_axis_name="c", subcore_axis_name="s")

  @functools.partial(
      pl.kernel,
      out_type=jax.ShapeDtypeStruct((_NC * _NPAD, _D_H), jnp.float32),
      mesh=mesh,
      scratch_types=[
          pltpu.VMEM((_NCHUNK, _CHUNK), jnp.int32),
          pltpu.VMEM((_NCHUNK, _CHUNK), jnp.int32),
          pltpu.VMEM((2 * _NB, _CHUNK, _D_H), jnp.float32),
          pltpu.VMEM_SHARED((_NPAD, _D_H), jnp.float32),
          pltpu.SemaphoreType.DMA,
          pltpu.SemaphoreType.DMA,
          pltpu.SemaphoreType.DMA,
          pltpu.SemaphoreType.DMA,
      ],
      compiler_params=_SC_PARAMS,
  )
  def k(z_hbm, s_hbm, d_hbm, out_hbm, sidx, didx, bufs, acc, gsA, gsB,
        ssA, ssB):
    cid = lax.axis_index("c")
    sid = lax.axis_index("s")
    wid = sid * _NC + cid
    r0 = sid * _RPT

    ic1 = pltpu.async_copy(
        s_hbm.at[pl.ds(wid * _NCHUNK, _NCHUNK)], sidx, gsB)
    ic2 = pltpu.async_copy(
        d_hbm.at[pl.ds(wid * _NCHUNK, _NCHUNK)], didx, gsB)

    zb = bufs.at[0]

    def zstore(i, carry):
      r = i // (_D_H // 16)
      c = (i % (_D_H // 16)) * 16
      zb[r, pl.ds(c, 16)] = jnp.zeros((16,), jnp.float32)
      return carry

    lax.fori_loop(0, _CHUNK * (_D_H // 16), zstore, 0)
    zcs = [
        pltpu.async_copy(zb, acc.at[pl.ds(r0 + j * _CHUNK, _CHUNK)], ssA)
        for j in range(_RPT // _CHUNK)
    ]
    for c in zcs:
      c.wait()
    ic1.wait()
    ic2.wait()
    plsc.subcore_barrier()

    def gfire(j, b, sem):
      pltpu.async_copy(z_hbm.at[sidx.at[j]], bufs.at[b], sem)

    def gwait(j, b, sem):
      pltpu.make_async_copy(z_hbm.at[sidx.at[j]], bufs.at[b], sem).wait()

    def sfire(j, b, sem):
      pltpu.async_copy(bufs.at[b], acc.at[didx.at[j]], sem, add=True)

    def swait(j, b, sem):
      pltpu.make_async_copy(bufs.at[b], acc.at[didx.at[j]], sem).wait()

    for b in range(_NB):
      gfire(b, b, gsA)

    def body(t, carry):
      jA = t * 2 * _NB
      jB = jA + _NB
      for b in range(_NB):
        gwait(jA + b, b, gsA)
      for b in range(_NB):
        gfire(jB + b, _NB + b, gsB)
      for b in range(_NB):
        sfire(jA + b, b, ssA)
      for b in range(_NB):
        swait(jA + b, b, ssA)

      @pl.when(t < _NCHUNK // (2 * _NB) - 1)
      def _():
        for b in range(_NB):
          gfire(jA + 2 * _NB + b, b, gsA)

      for b in range(_NB):
        gwait(jB + b, _NB + b, gsB)
      for b in range(_NB):
        sfire(jB + b, _NB + b, ssB)
      for b in range(_NB):
        swait(jB + b, _NB + b, ssB)
      return carry

    lax.fori_loop(0, _NCHUNK // (2 * _NB), body, 0)
    plsc.subcore_barrier()
    pltpu.sync_copy(acc.at[pl.ds(r0, _RPT)],
                    out_hbm.at[pl.ds(cid * _NPAD + r0, _RPT)])

  return k(z, src2d, dst2d)


_BR = 1000


def _tc_layer1(x, W1, deg0, deg1):
  def body(x_ref, w_ref, d0_ref, d1_ref, z_ref, dv_ref):
    deg = d0_ref[:, 0:1] + d1_ref[:, 0:1] + 1.0
    dinv = 1.0 / jnp.sqrt(deg)
    xw = jnp.dot(x_ref[...], w_ref[...], preferred_element_type=jnp.float32)
    z_ref[...] = dinv * xw
    dv_ref[...] = dinv

  return pl.pallas_call(
      body,
      grid=(_N // _BR,),
      in_specs=[
          pl.BlockSpec((_BR, _D_IN), lambda i: (i, 0)),
          pl.BlockSpec((_D_IN, _D_H), lambda i: (0, 0)),
          pl.BlockSpec((_BR, _DEG_W), lambda i: (i, 0)),
          pl.BlockSpec((_BR, _DEG_W), lambda i: (i, 0)),
      ],
      out_specs=[
          pl.BlockSpec((_BR, _D_H), lambda i: (i, 0)),
          pl.BlockSpec((_BR, 1), lambda i: (i, 0)),
      ],
      out_shape=[
          jax.ShapeDtypeStruct((_N, _D_H), jnp.float32),
          jax.ShapeDtypeStruct((_N, 1), jnp.float32),
      ],
  )(x, W1, deg0, deg1)


def _tc_mid(a0, a1, z1, dinv, b1, W2):
  def body(a0_ref, a1_ref, z_ref, dv_ref, b_ref, w_ref, z2_ref):
    dv = dv_ref[...]
    h = jnp.maximum(
        dv * (a0_ref[...] + a1_ref[...] + z_ref[...]) + b_ref[...], 0.0)
    z2_ref[...] = dv * jnp.dot(h, w_ref[...],
                               preferred_element_type=jnp.float32)

  return pl.pallas_call(
      body,
      grid=(_N // _BR,),
      in_specs=[
          pl.BlockSpec((_BR, _D_H), lambda i: (i, 0)),
          pl.BlockSpec((_BR, _D_H), lambda i: (i, 0)),
          pl.BlockSpec((_BR, _D_H), lambda i: (i, 0)),
          pl.BlockSpec((_BR, 1), lambda i: (i, 0)),
          pl.BlockSpec((1, _D_H), lambda i: (0, 0)),
          pl.BlockSpec((_D_H, _D_H), lambda i: (0, 0)),
      ],
      out_specs=pl.BlockSpec((_BR, _D_H), lambda i: (i, 0)),
      out_shape=jax.ShapeDtypeStruct((_N, _D_H), jnp.float32),
  )(a0, a1, z1, dinv, b1, W2)


def _tc_final(a0, a1, z2, dinv, b2):
  def body(a0_ref, a1_ref, z_ref, dv_ref, b_ref, o_ref):
    o_ref[...] = dv_ref[...] * (
        a0_ref[...] + a1_ref[...] + z_ref[...]) + b_ref[...]

  return pl.pallas_call(
      body,
      grid=(_N // _BR,),
      in_specs=[
          pl.BlockSpec((_BR, _D_H), lambda i: (i, 0)),
          pl.BlockSpec((_BR, _D_H), lambda i: (i, 0)),
          pl.BlockSpec((_BR, _D_H), lambda i: (i, 0)),
          pl.BlockSpec((_BR, 1), lambda i: (i, 0)),
          pl.BlockSpec((1, _D_H), lambda i: (0, 0)),
      ],
      out_specs=pl.BlockSpec((_BR, _D_H), lambda i: (i, 0)),
      out_shape=jax.ShapeDtypeStruct((_N, _D_H), jnp.float32),
  )(a0, a1, z2, dinv, b2)


def kernel(x, edge_index, W1, b1, W2, b2):
  src = edge_index[0]
  dst = edge_index[1]

  pad = _E_PAD - _E
  pidx = jnp.arange(pad, dtype=jnp.int32)
  srcp = jnp.concatenate([src, pidx % _N]).reshape(_E_PAD // _CHUNK, _CHUNK)
  dstp = jnp.concatenate(
      [dst, _N + pidx % (_NPAD - _N)]).reshape(_E_PAD // _CHUNK, _CHUNK)

  degp = _sc_degree(dstp).reshape(_NC, _NPAD, _DEG_W)
  z1, dinv = _tc_layer1(x, W1, degp[0, :_N], degp[1, :_N])

  acc1 = _sc_edge_pass(z1, srcp, dstp).reshape(_NC, _NPAD, _D_H)
  z2 = _tc_mid(acc1[0, :_N], acc1[1, :_N], z1, dinv,
               b1.reshape(1, _D_H), W2)

  acc2 = _sc_edge_pass(z2, srcp, dstp).reshape(_NC, _NPAD, _D_H)
  return _tc_final(acc2[0, :_N], acc2[1, :_N], z2, dinv,
                   b2.reshape(1, _D_H))


# speedup vs baseline: 42.8649x; 1.0917x over previous
import functools

import jax
import jax.numpy as jnp
from jax import lax
from jax.experimental import pallas as pl
from jax.experimental.pallas import tpu as pltpu
from jax.experimental.pallas import tpu_sc as plsc

_N = 10000
_E = 320000
_D_IN = 128
_D_H = 64

_NC = 2
_NS = 16
_NW = _NC * _NS
_CHUNK = 128

_NPAD = 10240
_RPT = _NPAD // _NS
_EPW = 10240
_E_PAD = _EPW * _NW
_NCHUNK = _EPW // _CHUNK
_DEG_W = 16
_NB = 4

_SC_PARAMS = pltpu.CompilerParams(use_tc_tiling_on_sc=False)


def _sc_degree(dst2d):
  mesh = plsc.VectorSubcoreMesh(core_axis_name="c", subcore_axis_name="s")

  @functools.partial(
      pl.kernel,
      out_type=jax.ShapeDtypeStruct((_NC * _N, _DEG_W), jnp.float32),
      mesh=mesh,
      scratch_types=[
          pltpu.VMEM((_NCHUNK, _CHUNK), jnp.int32),
          pltpu.VMEM((_CHUNK, _DEG_W), jnp.float32),
          pltpu.VMEM_SHARED((_NPAD, _DEG_W), jnp.float32),
          pltpu.SemaphoreType.DMA,
          pltpu.SemaphoreType.DMA,
      ],
      compiler_params=_SC_PARAMS,
  )
  def k(d_hbm, out_hbm, didx, buf, acc, isem, ssem):
    cid = lax.axis_index("c")
    sid = lax.axis_index("s")
    wid = sid * _NC + cid
    r0 = sid * _RPT

    idxc = pltpu.async_copy(
        d_hbm.at[pl.ds(wid * _NCHUNK, _NCHUNK)], didx, isem)

    def fill(i, val):
      buf[i] = jnp.full((_DEG_W,), val, jnp.float32)
      return val

    lax.fori_loop(0, _CHUNK, fill, 0.0)
    zcs = [
        pltpu.async_copy(buf, acc.at[pl.ds(r0 + j * _CHUNK, _CHUNK)], ssem)
        for j in range(_RPT // _CHUNK)
    ]
    tail = _RPT % _CHUNK
    if tail:
      zcs.append(pltpu.async_copy(
          buf.at[pl.ds(0, tail)],
          acc.at[pl.ds(r0 + (_RPT // _CHUNK) * _CHUNK, tail)], ssem))
    for c in zcs:
      c.wait()
    lax.fori_loop(0, _CHUNK, fill, 1.0)
    idxc.wait()
    plsc.subcore_barrier()

    def body(t, carry):
      cs = [
          pltpu.async_copy(buf, acc.at[didx.at[8 * t + b]], ssem, add=True)
          for b in range(8)
      ]
      for c in cs:
        c.wait()
      return carry

    lax.fori_loop(0, _NCHUNK // 8, body, 0)
    plsc.subcore_barrier()
    last = _N - (_NS - 1) * _RPT

    @pl.when(sid < _NS - 1)
    def _():
      pltpu.sync_copy(acc.at[pl.ds(r0, _RPT)],
                      out_hbm.at[pl.ds(cid * _N + r0, _RPT)])

    @pl.when(sid == _NS - 1)
    def _():
      pltpu.sync_copy(acc.at[pl.ds(r0, last)],
                      out_hbm.at[pl.ds(cid * _N + r0, last)])

  return k(dst2d)


def _sc_edge_pass(z, src2d, dst2d):
  mesh = plsc.VectorSubcoreMesh(core_axis_name="c", subcore_axis_name="s")

  @functools.partial(
      pl.kernel,
      out_type=jax.ShapeDtypeStruct((_NC * _N, _D_H), jnp.float32),
      mesh=mesh,
      scratch_types=[
          pltpu.VMEM((_NCHUNK, _CHUNK), jnp.int32),
          pltpu.VMEM((_NCHUNK, _CHUNK), jnp.int32),
          pltpu.VMEM((2 * _NB, _CHUNK, _D_H), jnp.float32),
          pltpu.VMEM_SHARED((_NPAD, _D_H), jnp.float32),
          pltpu.SemaphoreType.DMA,
          pltpu.SemaphoreType.DMA,
          pltpu.SemaphoreType.DMA,
          pltpu.SemaphoreType.DMA,
      ],
      compiler_params=_SC_PARAMS,
  )
  def k(z_hbm, s_hbm, d_hbm, out_hbm, sidx, didx, bufs, acc,
        gsA, gsB, ssA, ssB):
    cid = lax.axis_index("c")
    sid = lax.axis_index("s")
    wid = sid * _NC + cid
    r0 = sid * _RPT

    ic1 = pltpu.async_copy(
        s_hbm.at[pl.ds(wid * _NCHUNK, _NCHUNK)], sidx, gsB)
    ic2 = pltpu.async_copy(
        d_hbm.at[pl.ds(wid * _NCHUNK, _NCHUNK)], didx, gsB)

    zb = bufs.at[0]

    def zstore(i, carry):
      r = i // (_D_H // 16)
      c = (i % (_D_H // 16)) * 16
      zb[r, pl.ds(c, 16)] = jnp.zeros((16,), jnp.float32)
      return carry

    lax.fori_loop(0, _CHUNK * (_D_H // 16), zstore, 0)
    zcs = [
        pltpu.async_copy(zb, acc.at[pl.ds(r0 + j * _CHUNK, _CHUNK)], ssA)
        for j in range(_RPT // _CHUNK)
    ]
    tail = _RPT % _CHUNK
    if tail:
      zcs.append(pltpu.async_copy(
          zb.at[pl.ds(0, tail)],
          acc.at[pl.ds(r0 + (_RPT // _CHUNK) * _CHUNK, tail)], ssA))
    for c in zcs:
      c.wait()
    ic1.wait()
    ic2.wait()
    plsc.subcore_barrier()

    def gfire(j, b, sem):
      pltpu.async_copy(z_hbm.at[sidx.at[j]], bufs.at[b], sem)

    def gwait(j, b, sem):
      pltpu.make_async_copy(z_hbm.at[sidx.at[j]], bufs.at[b], sem).wait()

    def sfire(j, b, sem):
      pltpu.async_copy(bufs.at[b], acc.at[didx.at[j]], sem, add=True)

    def swait(j, b, sem):
      pltpu.make_async_copy(bufs.at[b], acc.at[didx.at[j]], sem).wait()

    for b in range(_NB):
      gfire(b, b, gsA)

    def body(t, carry):
      jA = t * 2 * _NB
      jB = jA + _NB
      for b in range(_NB):
        gwait(jA + b, b, gsA)
      for b in range(_NB):
        gfire(jB + b, _NB + b, gsB)
      for b in range(_NB):
        sfire(jA + b, b, ssA)
      for b in range(_NB):
        swait(jA + b, b, ssA)

      @pl.when(t < _NCHUNK // (2 * _NB) - 1)
      def _():
        for b in range(_NB):
          gfire(jA + 2 * _NB + b, b, gsA)

      for b in range(_NB):
        gwait(jB + b, _NB + b, gsB)
      for b in range(_NB):
        sfire(jB + b, _NB + b, ssB)
      for b in range(_NB):
        swait(jB + b, _NB + b, ssB)
      return carry

    lax.fori_loop(0, _NCHUNK // (2 * _NB), body, 0)
    plsc.subcore_barrier()
    last = _N - (_NS - 1) * _RPT

    @pl.when(sid < _NS - 1)
    def _():
      pltpu.sync_copy(acc.at[pl.ds(r0, _RPT)],
                      out_hbm.at[pl.ds(cid * _N + r0, _RPT)])

    @pl.when(sid == _NS - 1)
    def _():
      pltpu.sync_copy(acc.at[pl.ds(r0, last)],
                      out_hbm.at[pl.ds(cid * _N + r0, last)])

  return k(z, src2d, dst2d)


_BR = 1000


def _tc_layer1(x, W1, deg0, deg1):
  def body(x_ref, w_ref, d0_ref, d1_ref, z_ref, dv_ref):
    deg = d0_ref[:, 0:1] + d1_ref[:, 0:1] + 1.0
    dinv = 1.0 / jnp.sqrt(deg)
    xw = jnp.dot(x_ref[...], w_ref[...], preferred_element_type=jnp.float32)
    z_ref[...] = dinv * xw
    dv_ref[...] = dinv

  return pl.pallas_call(
      body,
      grid=(_N // _BR,),
      in_specs=[
          pl.BlockSpec((_BR, _D_IN), lambda i: (i, 0)),
          pl.BlockSpec((_D_IN, _D_H), lambda i: (0, 0)),
          pl.BlockSpec((_BR, _DEG_W), lambda i: (i, 0)),
          pl.BlockSpec((_BR, _DEG_W), lambda i: (i + _N // _BR, 0)),
      ],
      out_specs=[
          pl.BlockSpec((_BR, _D_H), lambda i: (i, 0)),
          pl.BlockSpec((_BR, 1), lambda i: (i, 0)),
      ],
      out_shape=[
          jax.ShapeDtypeStruct((_N, _D_H), jnp.float32),
          jax.ShapeDtypeStruct((_N, 1), jnp.float32),
      ],
  )(x, W1, deg0, deg1)


def _tc_mid(a0, a1, z1, dinv, b1, W2):
  def body(a0_ref, a1_ref, z_ref, dv_ref, b_ref, w_ref, z2_ref):
    dv = dv_ref[...]
    h = jnp.maximum(
        dv * (a0_ref[...] + a1_ref[...] + z_ref[...]) + b_ref[...], 0.0)
    z2_ref[...] = dv * jnp.dot(h, w_ref[...],
                               preferred_element_type=jnp.float32)

  return pl.pallas_call(
      body,
      grid=(_N // _BR,),
      in_specs=[
          pl.BlockSpec((_BR, _D_H), lambda i: (i, 0)),
          pl.BlockSpec((_BR, _D_H), lambda i: (i + _N // _BR, 0)),
          pl.BlockSpec((_BR, _D_H), lambda i: (i, 0)),
          pl.BlockSpec((_BR, 1), lambda i: (i, 0)),
          pl.BlockSpec((1, _D_H), lambda i: (0, 0)),
          pl.BlockSpec((_D_H, _D_H), lambda i: (0, 0)),
      ],
      out_specs=pl.BlockSpec((_BR, _D_H), lambda i: (i, 0)),
      out_shape=jax.ShapeDtypeStruct((_N, _D_H), jnp.float32),
  )(a0, a1, z1, dinv, b1, W2)


def _tc_final(a0, a1, z2, dinv, b2):
  def body(a0_ref, a1_ref, z_ref, dv_ref, b_ref, o_ref):
    o_ref[...] = dv_ref[...] * (
        a0_ref[...] + a1_ref[...] + z_ref[...]) + b_ref[...]

  return pl.pallas_call(
      body,
      grid=(_N // _BR,),
      in_specs=[
          pl.BlockSpec((_BR, _D_H), lambda i: (i, 0)),
          pl.BlockSpec((_BR, _D_H), lambda i: (i + _N // _BR, 0)),
          pl.BlockSpec((_BR, _D_H), lambda i: (i, 0)),
          pl.BlockSpec((_BR, 1), lambda i: (i, 0)),
          pl.BlockSpec((1, _D_H), lambda i: (0, 0)),
      ],
      out_specs=pl.BlockSpec((_BR, _D_H), lambda i: (i, 0)),
      out_shape=jax.ShapeDtypeStruct((_N, _D_H), jnp.float32),
  )(a0, a1, z2, dinv, b2)


def kernel(x, edge_index, W1, b1, W2, b2):
  src = edge_index[0]
  dst = edge_index[1]

  pad = _E_PAD - _E
  pidx = jnp.arange(pad, dtype=jnp.int32)
  srcp = jnp.concatenate([src, pidx % _N]).reshape(_E_PAD // _CHUNK, _CHUNK)
  dstp = jnp.concatenate(
      [dst, _N + pidx % (_NPAD - _N)]).reshape(_E_PAD // _CHUNK, _CHUNK)

  degp = _sc_degree(dstp)
  z1, dinv = _tc_layer1(x, W1, degp, degp)

  acc1 = _sc_edge_pass(z1, srcp, dstp)
  z2 = _tc_mid(acc1, acc1, z1, dinv, b1.reshape(1, _D_H), W2)

  acc2 = _sc_edge_pass(z2, srcp, dstp)
  return _tc_final(acc2, acc2, z2, dinv, b2.reshape(1, _D_H))
